# Initial kernel scaffold; baseline (speedup 1.0000x reference)
import jax
import jax.numpy as jnp
from jax.experimental import pallas as pl


def kernel(sequence, segment_label, token_table, pos_table, seg_table, gamma, beta):
    raise NotImplementedError("write your pallas kernel here")


# speedup vs baseline: 1.1614x; 1.1614x over previous
import functools

import jax
import jax.numpy as jnp
from jax import lax
from jax.experimental import pallas as pl
from jax.experimental.pallas import tpu as pltpu
from jax.experimental.pallas import tpu_sc as plsc

PAD = 0
EPS = 1e-5
NC = 2
NS = 16
LANES = 16
NW = NC * NS


def _take16(vec, idx16):
    return lax.gather(
        vec, idx16[:, None],
        lax.GatherDimensionNumbers(
            offset_dims=(), collapsed_slice_dims=(0,), start_index_map=(0,)),
        slice_sizes=(1,),
        mode=lax.GatherScatterMode.PROMISE_IN_BOUNDS)


def _lane_sum(x):
    iota = lax.iota(jnp.int32, LANES)
    for shift in (8, 4, 2, 1):
        x = x + _take16(x, (iota + shift) & (LANES - 1))
    return x


def _rsqrt(t):
    i = lax.bitcast_convert_type(t, jnp.int32)
    i = 0x5F3759DF - lax.shift_right_logical(i, 1)
    y = lax.bitcast_convert_type(i, jnp.float32)
    for _ in range(3):
        y = y * (1.5 - 0.5 * t * y * y)
    return y


def _make_sc_kernel(B, S, V, D, NSEG):
    SW = S // NW
    CS = 16
    C = B * CS
    NCHUNK = SW // CS
    ND16 = D // LANES

    mesh = plsc.VectorSubcoreMesh(core_axis_name="c", subcore_axis_name="s")

    @functools.partial(
        pl.kernel,
        out_type=jax.ShapeDtypeStruct((B * S, D), jnp.float32),
        mesh=mesh,
        scratch_types=[
            pltpu.VMEM((B * SW,), jnp.int32),
            pltpu.VMEM((B * SW,), jnp.int32),
            pltpu.VMEM((C,), jnp.int32),
            pltpu.VMEM((C, D), jnp.float32),
            pltpu.VMEM((SW, D), jnp.float32),
            pltpu.VMEM((NSEG * D,), jnp.float32),
            pltpu.VMEM((D,), jnp.float32),
            pltpu.VMEM((D,), jnp.float32),
            pltpu.VMEM((D,), jnp.float32),
            pltpu.SemaphoreType.DMA,
        ],
    )
    def sc_kernel(seq_hbm, lbl_hbm, tab_hbm, pos_hbm, seg_hbm, g_hbm, bt_hbm,
                  out_hbm, idx_v, lblv, cidx, tok_v, pos_v, seg_v, dseg_v,
                  gam_v, bet_v, sem):
        wid = lax.axis_index("s") * NC + lax.axis_index("c")
        s_base = wid * SW

        for b in range(B):
            pltpu.sync_copy(seq_hbm.at[pl.ds(b * S + s_base, SW)],
                            idx_v.at[pl.ds(b * SW, SW)])
            pltpu.sync_copy(lbl_hbm.at[pl.ds(b * S + s_base, SW)],
                            lblv.at[pl.ds(b * SW, SW)])
        pltpu.sync_copy(pos_hbm.at[pl.ds(s_base, SW)], pos_v)
        pltpu.sync_copy(seg_hbm, seg_v)
        pltpu.sync_copy(g_hbm, gam_v)
        pltpu.sync_copy(bt_hbm, bet_v)
        for k in range(ND16):
            dsl = pl.ds(k * LANES, LANES)
            dseg_v[dsl] = seg_v[pl.ds(D + k * LANES, LANES)] - seg_v[dsl]

        def chunk_body(ci, _):
            for b in range(B):
                for k in range(CS // LANES):
                    sl = pl.ds(b * SW + ci * CS + k * LANES, LANES)
                    cidx[pl.ds(b * CS + k * LANES, LANES)] = idx_v[sl]
            pltpu.async_copy(tab_hbm.at[cidx], tok_v, sem).wait()

            def tok_body(j, _):
                b = j // CS
                si = j % CS
                srow = ci * CS + si
                si_splat = jnp.full((LANES,), si, jnp.int32)
                idx_vec = cidx[pl.ds(b * CS, LANES)]
                m_vec = jnp.where(idx_vec != PAD, 1.0, 0.0)
                m = _take16(m_vec, si_splat)
                lbl_vec = lblv[pl.ds(b * SW + ci * CS, LANES)]
                lbf = _take16(lbl_vec.astype(jnp.float32), si_splat)

                def d_body(k, carry):
                    acc, acc2 = carry
                    dsl = pl.ds(k * LANES, LANES)
                    seg_x = seg_v[dsl] + lbf * dseg_v[dsl]
                    x = (tok_v[j, dsl] + pos_v[srow, dsl]) * m + seg_x
                    tok_v[j, dsl] = x
                    return acc + x, acc2 + x * x

                acc, acc2 = lax.fori_loop(
                    0, ND16, d_body,
                    (jnp.zeros((LANES,), jnp.float32),
                     jnp.zeros((LANES,), jnp.float32)))
                mean = _lane_sum(acc) * (1.0 / D)
                msq = _lane_sum(acc2) * (1.0 / D)
                rinv = _rsqrt(msq - mean * mean + EPS)

                def d2_body(k, _):
                    dsl = pl.ds(k * LANES, LANES)
                    y = (tok_v[j, dsl] - mean) * rinv * gam_v[dsl] + bet_v[dsl]
                    tok_v[j, dsl] = y
                    return 0

                lax.fori_loop(0, ND16, d2_body, 0)
                return 0

            lax.fori_loop(0, C, tok_body, 0)

            for b in range(B):
                pltpu.sync_copy(
                    tok_v.at[pl.ds(b * CS, CS)],
                    out_hbm.at[pl.ds(b * S + s_base + ci * CS, CS)])
            return 0

        lax.fori_loop(0, NCHUNK, chunk_body, 0)

    return sc_kernel


def kernel(sequence, segment_label, token_table, pos_table, seg_table, gamma,
           beta):
    B, S = sequence.shape
    V, D = token_table.shape
    NSEG = seg_table.shape[0]
    seq = sequence.astype(jnp.int32).reshape(-1)
    lbl = segment_label.astype(jnp.int32).reshape(-1)
    sc = _make_sc_kernel(B, S, V, D, NSEG)
    out = sc(seq, lbl, token_table, pos_table, seg_table.reshape(-1), gamma,
             beta)
    return out.reshape(B, S, D)


# speedup vs baseline: 1.3022x; 1.1213x over previous
import functools

import jax
import jax.numpy as jnp
from jax import lax
from jax.experimental import pallas as pl
from jax.experimental.pallas import tpu as pltpu
from jax.experimental.pallas import tpu_sc as plsc

PAD = 0
EPS = 1e-5
NC = 2
NS = 16
LANES = 16
NW = NC * NS


def _take16(vec, idx16):
    return lax.gather(
        vec, idx16[:, None],
        lax.GatherDimensionNumbers(
            offset_dims=(), collapsed_slice_dims=(0,), start_index_map=(0,)),
        slice_sizes=(1,),
        mode=lax.GatherScatterMode.PROMISE_IN_BOUNDS)


def _lane_sum(x):
    iota = lax.iota(jnp.int32, LANES)
    for shift in (8, 4, 2, 1):
        x = x + _take16(x, (iota + shift) & (LANES - 1))
    return x


def _rsqrt(t):
    i = lax.bitcast_convert_type(t, jnp.int32)
    i = 0x5F3759DF - lax.shift_right_logical(i, 1)
    y = lax.bitcast_convert_type(i, jnp.float32)
    for _ in range(3):
        y = y * (1.5 - 0.5 * t * y * y)
    return y


def _make_sc_kernel(B, S, V, D, NSEG):
    SW = S // NW
    CS = 16
    C = B * CS
    NCHUNK = SW // CS
    ND16 = D // LANES

    mesh = plsc.VectorSubcoreMesh(core_axis_name="c", subcore_axis_name="s")

    @functools.partial(
        pl.kernel,
        out_type=jax.ShapeDtypeStruct((B * S, D), jnp.float32),
        mesh=mesh,
        scratch_types=[
            pltpu.VMEM((B * SW,), jnp.int32),
            pltpu.VMEM((B * SW,), jnp.int32),
            pltpu.VMEM((C,), jnp.int32),
            pltpu.VMEM((C, D), jnp.float32),
            pltpu.VMEM((SW, D), jnp.float32),
            pltpu.VMEM((NSEG * D,), jnp.float32),
            pltpu.VMEM((D,), jnp.float32),
            pltpu.VMEM((D,), jnp.float32),
            pltpu.VMEM((D,), jnp.float32),
            pltpu.SemaphoreType.DMA,
        ],
    )
    def sc_kernel(seq_hbm, lbl_hbm, tab_hbm, pos_hbm, seg_hbm, g_hbm, bt_hbm,
                  out_hbm, idx_v, lblv, cidx, tok_v, pos_v, seg_v, dseg_v,
                  gam_v, bet_v, sem):
        wid = lax.axis_index("s") * NC + lax.axis_index("c")
        s_base = wid * SW

        for b in range(B):
            pltpu.sync_copy(seq_hbm.at[pl.ds(b * S + s_base, SW)],
                            idx_v.at[pl.ds(b * SW, SW)])
            pltpu.sync_copy(lbl_hbm.at[pl.ds(b * S + s_base, SW)],
                            lblv.at[pl.ds(b * SW, SW)])
        pltpu.sync_copy(pos_hbm.at[pl.ds(s_base, SW)], pos_v)
        pltpu.sync_copy(seg_hbm, seg_v)
        pltpu.sync_copy(g_hbm, gam_v)
        pltpu.sync_copy(bt_hbm, bet_v)
        for k in range(ND16):
            dsl = pl.ds(k * LANES, LANES)
            dseg_v[dsl] = seg_v[pl.ds(D + k * LANES, LANES)] - seg_v[dsl]

        def chunk_body(ci, _):
            for b in range(B):
                for k in range(CS // LANES):
                    sl = pl.ds(b * SW + ci * CS + k * LANES, LANES)
                    cidx[pl.ds(b * CS + k * LANES, LANES)] = idx_v[sl]
            pltpu.async_copy(tab_hbm.at[cidx], tok_v, sem).wait()

            def si_body(si, _):
                srow = ci * CS + si
                si_splat = jnp.full((LANES,), si, jnp.int32)
                ms = []
                lbfs = []
                for b in range(B):
                    idx_vec = cidx[pl.ds(b * CS, LANES)]
                    m_vec = jnp.where(idx_vec != PAD, 1.0, 0.0)
                    ms.append(_take16(m_vec, si_splat))
                    lbl_vec = lblv[pl.ds(b * SW + ci * CS, LANES)]
                    lbfs.append(_take16(lbl_vec.astype(jnp.float32),
                                        si_splat))

                def d_body(k, carry):
                    accs = carry
                    dsl = pl.ds(k * LANES, LANES)
                    pos_x = pos_v[srow, dsl]
                    seg0_x = seg_v[dsl]
                    dseg_x = dseg_v[dsl]
                    out = []
                    for b in range(B):
                        segc = seg0_x + lbfs[b] * dseg_x
                        x = (tok_v[b * CS + si, dsl] + pos_x) * ms[b] + segc
                        tok_v[b * CS + si, dsl] = x
                        out.append((accs[b][0] + x, accs[b][1] + x * x))
                    return tuple(out)

                zz = jnp.zeros((LANES,), jnp.float32)
                accs = lax.fori_loop(0, ND16, d_body,
                                     tuple((zz, zz) for _ in range(B)),
                                     unroll=4)
                means = []
                rinvs = []
                for b in range(B):
                    mean = _lane_sum(accs[b][0]) * (1.0 / D)
                    msq = _lane_sum(accs[b][1]) * (1.0 / D)
                    means.append(mean)
                    rinvs.append(_rsqrt(msq - mean * mean + EPS))

                def d2_body(k, _):
                    dsl = pl.ds(k * LANES, LANES)
                    g_x = gam_v[dsl]
                    b_x = bet_v[dsl]
                    for b in range(B):
                        y = ((tok_v[b * CS + si, dsl] - means[b]) * rinvs[b]
                             * g_x + b_x)
                        tok_v[b * CS + si, dsl] = y
                    return 0

                lax.fori_loop(0, ND16, d2_body, 0, unroll=4)
                return 0

            lax.fori_loop(0, CS, si_body, 0)

            for b in range(B):
                pltpu.sync_copy(
                    tok_v.at[pl.ds(b * CS, CS)],
                    out_hbm.at[pl.ds(b * S + s_base + ci * CS, CS)])
            return 0

        lax.fori_loop(0, NCHUNK, chunk_body, 0)

    return sc_kernel


def kernel(sequence, segment_label, token_table, pos_table, seg_table, gamma,
           beta):
    B, S = sequence.shape
    V, D = token_table.shape
    NSEG = seg_table.shape[0]
    seq = sequence.astype(jnp.int32).reshape(-1)
    lbl = segment_label.astype(jnp.int32).reshape(-1)
    sc = _make_sc_kernel(B, S, V, D, NSEG)
    out = sc(seq, lbl, token_table, pos_table, seg_table.reshape(-1), gamma,
             beta)
    return out.reshape(B, S, D)


# speedup vs baseline: 4.6145x; 3.5435x over previous
import functools

import jax
import jax.numpy as jnp
from jax import lax
from jax.experimental import pallas as pl
from jax.experimental.pallas import tpu as pltpu
from jax.experimental.pallas import tpu_sc as plsc

PAD = 0
EPS = 1e-5
NC = 2
NS = 16
NW = NC * NS


def _make_sc_gather(N, V, D):
    TPW = N // NW
    CR = 64
    NCH = TPW // CR

    mesh = plsc.VectorSubcoreMesh(core_axis_name="c", subcore_axis_name="s")

    @functools.partial(
        pl.kernel,
        out_type=jax.ShapeDtypeStruct((N, D), jnp.float32),
        mesh=mesh,
        scratch_types=[
            pltpu.VMEM((TPW,), jnp.int32),
            pltpu.VMEM((CR, D), jnp.float32),
            pltpu.VMEM((CR, D), jnp.float32),
            pltpu.SemaphoreType.DMA,
            pltpu.SemaphoreType.DMA,
            pltpu.SemaphoreType.DMA,
            pltpu.SemaphoreType.DMA,
        ],
    )
    def sc_gather(idx_hbm, tab_hbm, out_hbm, idx_v, buf0, buf1,
                  sg0, sg1, so0, so1):
        wid = lax.axis_index("s") * NC + lax.axis_index("c")
        base = wid * TPW
        pltpu.sync_copy(idx_hbm.at[pl.ds(base, TPW)], idx_v)

        bufs = (buf0, buf1)
        sgs = (sg0, sg1)
        sos = (so0, so1)
        gh = {}
        oh = {}
        gh[0] = pltpu.async_copy(
            tab_hbm.at[idx_v.at[pl.ds(0, CR)]], bufs[0], sgs[0])
        for ci in range(NCH):
            gh[ci].wait()
            if ci + 1 < NCH:
                if ci - 1 >= 0:
                    oh[ci - 1].wait()
                gh[ci + 1] = pltpu.async_copy(
                    tab_hbm.at[idx_v.at[pl.ds((ci + 1) * CR, CR)]],
                    bufs[(ci + 1) % 2], sgs[(ci + 1) % 2])
            oh[ci] = pltpu.async_copy(
                bufs[ci % 2], out_hbm.at[pl.ds(base + ci * CR, CR)],
                sos[ci % 2])
        oh[NCH - 2].wait()
        oh[NCH - 1].wait()

    return sc_gather


def _make_tc_ln(B, S, D, NSEG):
    TS = 512
    NSB = S // TS

    def body(seq_ref, lbl_ref, tok_ref, pos_ref, seg_ref, gam_ref, bet_ref,
             o_ref):
        b = pl.program_id(1)
        tok = tok_ref[0]
        pos = pos_ref[...]
        m = (seq_ref[b] != PAD).astype(jnp.float32)[:, None]
        lbf = lbl_ref[b].astype(jnp.float32)[:, None]
        seg0 = seg_ref[0:1, :]
        dseg = seg_ref[1:2, :] - seg0
        x = (tok + pos) * m + seg0 + lbf * dseg
        mean = jnp.mean(x, axis=1, keepdims=True)
        xc = x - mean
        var = jnp.mean(xc * xc, axis=1, keepdims=True)
        o_ref[0] = xc * lax.rsqrt(var + EPS) * gam_ref[...] + bet_ref[...]

    return pl.pallas_call(
        body,
        grid=(NSB, B),
        in_specs=[
            pl.BlockSpec((B, TS), lambda i, j: (0, i)),
            pl.BlockSpec((B, TS), lambda i, j: (0, i)),
            pl.BlockSpec((1, TS, D), lambda i, j: (j, i, 0)),
            pl.BlockSpec((TS, D), lambda i, j: (i, 0)),
            pl.BlockSpec((NSEG, D), lambda i, j: (0, 0)),
            pl.BlockSpec((1, D), lambda i, j: (0, 0)),
            pl.BlockSpec((1, D), lambda i, j: (0, 0)),
        ],
        out_specs=pl.BlockSpec((1, TS, D), lambda i, j: (j, i, 0)),
        out_shape=jax.ShapeDtypeStruct((B, S, D), jnp.float32),
    )


def kernel(sequence, segment_label, token_table, pos_table, seg_table, gamma,
           beta):
    B, S = sequence.shape
    V, D = token_table.shape
    NSEG = seg_table.shape[0]
    seq = sequence.astype(jnp.int32)
    lbl = segment_label.astype(jnp.int32)

    gathered = _make_sc_gather(B * S, V, D)(seq.reshape(-1), token_table)
    out = _make_tc_ln(B, S, D, NSEG)(
        seq, lbl, gathered.reshape(B, S, D), pos_table[:S],
        seg_table, gamma.reshape(1, D), beta.reshape(1, D))
    return out


# speedup vs baseline: 4.6404x; 1.0056x over previous
import functools

import jax
import jax.numpy as jnp
from jax import lax
from jax.experimental import pallas as pl
from jax.experimental.pallas import tpu as pltpu
from jax.experimental.pallas import tpu_sc as plsc

PAD = 0
EPS = 1e-5
NC = 2
NS = 16
NW = NC * NS
SPLIT = 4
TS = 512
CR = 64


def _make_sc_gather(N, V, D):
    TPW = N // NW
    NCH = max(TPW // CR, 1)
    CRW = min(TPW, CR)

    mesh = plsc.VectorSubcoreMesh(core_axis_name="c", subcore_axis_name="s")

    @functools.partial(
        pl.kernel,
        out_type=jax.ShapeDtypeStruct((N, D), jnp.float32),
        mesh=mesh,
        scratch_types=[
            pltpu.VMEM((TPW,), jnp.int32),
            pltpu.VMEM((CRW, D), jnp.float32),
            pltpu.VMEM((CRW, D), jnp.float32),
            pltpu.SemaphoreType.DMA,
            pltpu.SemaphoreType.DMA,
            pltpu.SemaphoreType.DMA,
            pltpu.SemaphoreType.DMA,
        ],
    )
    def sc_gather(idx_hbm, tab_hbm, out_hbm, idx_v, buf0, buf1,
                  sg0, sg1, so0, so1):
        wid = lax.axis_index("s") * NC + lax.axis_index("c")
        base = wid * TPW
        pltpu.sync_copy(idx_hbm.at[pl.ds(base, TPW)], idx_v)

        bufs = (buf0, buf1)
        sgs = (sg0, sg1)
        sos = (so0, so1)
        gh = {}
        oh = {}
        gh[0] = pltpu.async_copy(
            tab_hbm.at[idx_v.at[pl.ds(0, CRW)]], bufs[0], sgs[0])
        for ci in range(NCH):
            gh[ci].wait()
            if ci + 1 < NCH:
                if ci - 1 >= 0:
                    oh[ci - 1].wait()
                gh[ci + 1] = pltpu.async_copy(
                    tab_hbm.at[idx_v.at[pl.ds((ci + 1) * CRW, CRW)]],
                    bufs[(ci + 1) % 2], sgs[(ci + 1) % 2])
            oh[ci] = pltpu.async_copy(
                bufs[ci % 2], out_hbm.at[pl.ds(base + ci * CRW, CRW)],
                sos[ci % 2])
        for ci in range(max(NCH - 2, 0), NCH):
            oh[ci].wait()

    return sc_gather


def _tc_ln_piece(B, S, D, NSEG, p, seq, lbl, tok_p, pos, seg, gam, bet,
                 carry):
    SP = S // SPLIT
    NSBP = SP // TS
    i0 = p * NSBP

    def body(seq_ref, lbl_ref, tok_ref, pos_ref, seg_ref, gam_ref, bet_ref,
             *rest):
        o_ref = rest[-1]
        b = pl.program_id(1)
        tok = tok_ref[0]
        pos_x = pos_ref[...]
        m = (seq_ref[b] != PAD).astype(jnp.float32)[:, None]
        lbf = lbl_ref[b].astype(jnp.float32)[:, None]
        seg0 = seg_ref[0:1, :]
        dseg = seg_ref[1:2, :] - seg0
        x = (tok + pos_x) * m + seg0 + lbf * dseg
        mean = jnp.mean(x, axis=1, keepdims=True)
        xc = x - mean
        var = jnp.mean(xc * xc, axis=1, keepdims=True)
        o_ref[0] = xc * lax.rsqrt(var + EPS) * gam_ref[...] + bet_ref[...]

    in_specs = [
        pl.BlockSpec((B, TS), lambda i, j: (0, i0 + i)),
        pl.BlockSpec((B, TS), lambda i, j: (0, i0 + i)),
        pl.BlockSpec((1, TS, D), lambda i, j: (j, i, 0)),
        pl.BlockSpec((TS, D), lambda i, j: (i0 + i, 0)),
        pl.BlockSpec((NSEG, D), lambda i, j: (0, 0)),
        pl.BlockSpec((1, D), lambda i, j: (0, 0)),
        pl.BlockSpec((1, D), lambda i, j: (0, 0)),
    ]
    args = [seq, lbl, tok_p, pos, seg, gam, bet]
    aliases = {}
    if carry is not None:
        in_specs.append(pl.BlockSpec(memory_space=pl.ANY))
        args.append(carry)
        aliases = {7: 0}

    return pl.pallas_call(
        body,
        grid=(NSBP, B),
        in_specs=in_specs,
        out_specs=pl.BlockSpec((1, TS, D), lambda i, j: (j, i0 + i, 0)),
        out_shape=jax.ShapeDtypeStruct((B, S, D), jnp.float32),
        input_output_aliases=aliases,
    )(*args)


def kernel(sequence, segment_label, token_table, pos_table, seg_table, gamma,
           beta):
    B, S = sequence.shape
    V, D = token_table.shape
    NSEG = seg_table.shape[0]
    SP = S // SPLIT
    seq = sequence.astype(jnp.int32)
    lbl = segment_label.astype(jnp.int32)
    gam = gamma.reshape(1, D)
    bet = beta.reshape(1, D)
    pos = pos_table[:S]

    sc_gather = _make_sc_gather(B * SP, V, D)
    gathered = [
        sc_gather(lax.slice(seq, (0, p * SP), (B, (p + 1) * SP))
                  .reshape(-1), token_table).reshape(B, SP, D)
        for p in range(SPLIT)
    ]

    out = None
    for p in range(SPLIT):
        out = _tc_ln_piece(B, S, D, NSEG, p, seq, lbl, gathered[p], pos,
                           seg_table, gam, bet, out)
    return out
